# baseline (speedup 1.0000x reference)
import jax
import jax.numpy as jnp
from jax.experimental import pallas as pl
from jax.experimental.pallas import tpu as pltpu


def _dinv_from_rowsum(s):
    d = s + 1.0
    d = jnp.where(d <= 1e-4, jnp.float32(1e-4), d)
    return jax.lax.rsqrt(d)


def kernel(X, A, W):
    B, N, C_IN = X.shape
    C_OUT = W.shape[1]
    F = B * C_OUT

    BR = 512
    BC = 2048
    nr = N // BR
    ncb = N // BC

    fb = [((r + 1) * BR) // BC for r in range(nr)]
    cnt2 = [ncb - fb[r] for r in range(nr)]
    rows_l, cbs_l, first_l = [], [], []
    for r in range(nr):
        for i in range(cnt2[r]):
            rows_l.append(r)
            cbs_l.append(fb[r] + i)
            first_l.append(1 if i == 0 else 0)
    nsteps2 = len(rows_l)
    nvis = sum(1 for r in range(nr) if cnt2[r] > 0)
    r_tab = jnp.asarray(rows_l, dtype=jnp.int32)
    cb_tab = jnp.asarray(cbs_l, dtype=jnp.int32)
    first_tab = jnp.asarray(first_l, dtype=jnp.int32)

    Xr = jnp.transpose(X, (1, 0, 2)).reshape(N, B * C_IN)

    def sweep(x_ref, w_ref, *rest):
        a_refs = rest[:ncb]
        p_ref, dinv_ref, z_ref, zsc = rest[ncb:]
        r = pl.program_id(0)

        s = jnp.sum(a_refs[0][...], axis=1, keepdims=True)
        for q in range(1, ncb):
            s = s + jnp.sum(a_refs[q][...], axis=1, keepdims=True)
        dinv = _dinv_from_rowsum(s)
        x = x_ref[...]
        y = jnp.dot(x.reshape(-1, C_IN), w_ref[...],
                    preferred_element_type=jnp.float32).reshape(x.shape[0], -1)
        z = dinv * y
        dinv_ref[...] = dinv
        z_ref[...] = z
        zsc[pl.ds(r * BR, BR), :] = z

        row_ids = jax.lax.broadcasted_iota(jnp.int32, (N, F), 0)
        zfull = jnp.where(row_ids < (r + 1) * BR, zsc[...], 0.0)
        acc = jnp.dot(a_refs[0][...], zfull[0:BC],
                      preferred_element_type=jnp.float32)
        for q in range(1, ncb):
            acc = acc + jnp.dot(a_refs[q][...], zfull[q * BC:(q + 1) * BC],
                                preferred_element_type=jnp.float32)
        p_ref[...] = acc

    P, Dinv, Z = pl.pallas_call(
        sweep,
        grid=(nr,),
        in_specs=[
            pl.BlockSpec((BR, B * C_IN), lambda r: (r, 0)),
            pl.BlockSpec((C_IN, C_OUT), lambda r: (0, 0)),
        ] + [
            pl.BlockSpec((BR, BC), (lambda r, q=q: (r, q)))
            for q in range(ncb)
        ],
        out_specs=[
            pl.BlockSpec((BR, F), lambda r: (r, 0)),
            pl.BlockSpec((BR, 1), lambda r: (r, 0)),
            pl.BlockSpec((BR, F), lambda r: (r, 0)),
        ],
        out_shape=[
            jax.ShapeDtypeStruct((N, F), jnp.float32),
            jax.ShapeDtypeStruct((N, 1), jnp.float32),
            jax.ShapeDtypeStruct((N, F), jnp.float32),
        ],
        scratch_shapes=[pltpu.VMEM((N, F), jnp.float32)],
        compiler_params=pltpu.CompilerParams(
            dimension_semantics=("arbitrary",),
        ),
    )(Xr, W, *([A] * ncb))

    NSUB = 4
    SW = BC // NSUB

    def upper(rt, ct, ft, zc_ref, *refs):
        k = pl.program_id(0)
        a_subs = refs[:NSUB]
        o_ref = refs[NSUB]
        r = rt[k]
        cb = ct[k]
        zc = zc_ref[...]
        col_ids = jax.lax.broadcasted_iota(jnp.int32, zc.shape, 0) + cb * BC
        zm = jnp.where(col_ids >= (r + 1) * BR, zc, 0.0)
        part = jnp.dot(a_subs[0][...], zm[0:SW],
                       preferred_element_type=jnp.float32)
        for i in range(1, NSUB):
            part = part + jnp.dot(a_subs[i][...], zm[i * SW:(i + 1) * SW],
                                  preferred_element_type=jnp.float32)

        @pl.when(ft[k] == 1)
        def _first():
            o_ref[...] = part

        @pl.when(ft[k] != 1)
        def _acc():
            o_ref[...] = o_ref[...] + part

    Oup = pl.pallas_call(
        upper,
        grid_spec=pltpu.PrefetchScalarGridSpec(
            num_scalar_prefetch=3,
            grid=(nsteps2,),
            in_specs=[
                pl.BlockSpec((BC, F), lambda k, rt, ct, ft: (ct[k], 0)),
            ] + [
                pl.BlockSpec(
                    (BR, SW),
                    (lambda k, rt, ct, ft, i=i: (rt[k], ct[k] * NSUB + i)))
                for i in range(NSUB)
            ],
            out_specs=pl.BlockSpec((BR, F), lambda k, rt, ct, ft: (rt[k], 0)),
        ),
        out_shape=jax.ShapeDtypeStruct((N, F), jnp.float32),
        compiler_params=pltpu.CompilerParams(
            dimension_semantics=("arbitrary",),
        ),
    )(r_tab, cb_tab, first_tab, Z, *([A] * NSUB))

    nv = nvis * BR
    acc = jnp.concatenate([P[:nv] + Oup[:nv], P[nv:]], axis=0)
    out2 = Dinv * acc + Dinv * Z
    return out2.reshape(N, B, C_OUT).transpose(1, 0, 2)


# speedup vs baseline: 1.0168x; 1.0168x over previous
import jax
import jax.numpy as jnp
from jax.experimental import pallas as pl
from jax.experimental.pallas import tpu as pltpu


def _dinv_from_rowsum(s):
    d = s + 1.0
    d = jnp.where(d <= 1e-4, jnp.float32(1e-4), d)
    return jax.lax.rsqrt(d)


def kernel(X, A, W):
    B, N, C_IN = X.shape
    C_OUT = W.shape[1]
    F = B * C_OUT

    BR = 512
    BC = 2048
    nr = N // BR
    ncb = N // BC

    fb = [((r + 1) * BR) // BC for r in range(nr)]
    cnt2 = [ncb - fb[r] for r in range(nr)]
    rows_l, cbs_l, first_l = [], [], []
    for r in range(nr):
        for i in range(cnt2[r]):
            rows_l.append(r)
            cbs_l.append(fb[r] + i)
            first_l.append(1 if i == 0 else 0)
    nsteps2 = len(rows_l)
    nvis = sum(1 for r in range(nr) if cnt2[r] > 0)
    r_tab = jnp.asarray(rows_l, dtype=jnp.int32)
    cb_tab = jnp.asarray(cbs_l, dtype=jnp.int32)
    first_tab = jnp.asarray(first_l, dtype=jnp.int32)

    Xr = jnp.transpose(X, (1, 0, 2)).reshape(N, B * C_IN)

    def sweep(x_ref, w_ref, a_ref, p_ref, dinv_ref, z_ref, zsc):
        r = pl.program_id(0)

        s = jnp.sum(a_ref[...], axis=1, keepdims=True)
        dinv = _dinv_from_rowsum(s)
        x = x_ref[...]
        y = jnp.dot(x.reshape(-1, C_IN), w_ref[...],
                    preferred_element_type=jnp.float32).reshape(x.shape[0], -1)
        z = dinv * y
        dinv_ref[...] = dinv
        z_ref[...] = z
        zsc[pl.ds(r * BR, BR), :] = z

        row_ids = jax.lax.broadcasted_iota(jnp.int32, (N, F), 0)
        zfull = jnp.where(row_ids < (r + 1) * BR, zsc[...], 0.0)
        p_ref[...] = jnp.dot(a_ref[...], zfull,
                             preferred_element_type=jnp.float32)

    P, Dinv, Z = pl.pallas_call(
        sweep,
        grid=(nr,),
        in_specs=[
            pl.BlockSpec((BR, B * C_IN), lambda r: (r, 0)),
            pl.BlockSpec((C_IN, C_OUT), lambda r: (0, 0)),
            pl.BlockSpec((BR, N), lambda r: (r, 0)),
        ],
        out_specs=[
            pl.BlockSpec((BR, F), lambda r: (r, 0)),
            pl.BlockSpec((BR, 1), lambda r: (r, 0)),
            pl.BlockSpec((BR, F), lambda r: (r, 0)),
        ],
        out_shape=[
            jax.ShapeDtypeStruct((N, F), jnp.float32),
            jax.ShapeDtypeStruct((N, 1), jnp.float32),
            jax.ShapeDtypeStruct((N, F), jnp.float32),
        ],
        scratch_shapes=[pltpu.VMEM((N, F), jnp.float32)],
        compiler_params=pltpu.CompilerParams(
            dimension_semantics=("arbitrary",),
        ),
    )(Xr, W, A)

    def upper(rt, ct, ft, zc_ref, a_ref, o_ref):
        k = pl.program_id(0)
        r = rt[k]
        cb = ct[k]
        zc = zc_ref[...]
        col_ids = jax.lax.broadcasted_iota(jnp.int32, zc.shape, 0) + cb * BC
        zm = jnp.where(col_ids >= (r + 1) * BR, zc, 0.0)
        part = jnp.dot(a_ref[...], zm, preferred_element_type=jnp.float32)

        @pl.when(ft[k] == 1)
        def _first():
            o_ref[...] = part

        @pl.when(ft[k] != 1)
        def _acc():
            o_ref[...] = o_ref[...] + part

    Oup = pl.pallas_call(
        upper,
        grid_spec=pltpu.PrefetchScalarGridSpec(
            num_scalar_prefetch=3,
            grid=(nsteps2,),
            in_specs=[
                pl.BlockSpec((BC, F), lambda k, rt, ct, ft: (ct[k], 0)),
                pl.BlockSpec((BR, BC), lambda k, rt, ct, ft: (rt[k], ct[k])),
            ],
            out_specs=pl.BlockSpec((BR, F), lambda k, rt, ct, ft: (rt[k], 0)),
        ),
        out_shape=jax.ShapeDtypeStruct((N, F), jnp.float32),
        compiler_params=pltpu.CompilerParams(
            dimension_semantics=("arbitrary",),
        ),
    )(r_tab, cb_tab, first_tab, Z, A)

    nv = nvis * BR
    acc = jnp.concatenate([P[:nv] + Oup[:nv], P[nv:]], axis=0)
    out2 = Dinv * acc + Dinv * Z
    return out2.reshape(N, B, C_OUT).transpose(1, 0, 2)


# speedup vs baseline: 1.0959x; 1.0778x over previous
import jax
import jax.numpy as jnp
from jax.experimental import pallas as pl
from jax.experimental.pallas import tpu as pltpu


def _dinv_from_rowsum(s):
    d = s + 1.0
    d = jnp.where(d <= 1e-4, jnp.float32(1e-4), d)
    return jax.lax.rsqrt(d)


def kernel(X, A, W):
    B, N, C_IN = X.shape
    C_OUT = W.shape[1]
    F = B * C_OUT

    BR = 512
    BC = 2048
    nr = N // BR
    ncb = N // BC

    fb = [((r + 1) * BR) // BC for r in range(nr)]
    cnt2 = [ncb - fb[r] for r in range(nr)]
    rows_l, cbs_l, first_l, last_l = [], [], [], []
    for r in range(nr):
        for i in range(cnt2[r]):
            rows_l.append(r)
            cbs_l.append(fb[r] + i)
            first_l.append(1 if i == 0 else 0)
            last_l.append(1 if i == cnt2[r] - 1 else 0)
    nsteps2 = len(rows_l)
    nvis = sum(1 for r in range(nr) if cnt2[r] > 0)
    r_tab = jnp.asarray(rows_l, dtype=jnp.int32)
    cb_tab = jnp.asarray(cbs_l, dtype=jnp.int32)
    first_tab = jnp.asarray(first_l, dtype=jnp.int32)
    last_tab = jnp.asarray(last_l, dtype=jnp.int32)

    Xr = jnp.transpose(X, (1, 0, 2)).reshape(N, B * C_IN)

    def sweep(x_ref, w_ref, a_ref, p_ref, dinv_ref, z_ref, zsc):
        r = pl.program_id(0)

        s = jnp.sum(a_ref[...], axis=1, keepdims=True)
        dinv = _dinv_from_rowsum(s)
        x = x_ref[...]
        y = jnp.dot(x.reshape(-1, C_IN), w_ref[...],
                    preferred_element_type=jnp.float32).reshape(x.shape[0], -1)
        z = dinv * y
        dinv_ref[...] = dinv
        z_ref[...] = z
        zsc[pl.ds(r * BR, BR), :] = z

        row_ids = jax.lax.broadcasted_iota(jnp.int32, (N, F), 0)
        zfull = jnp.where(row_ids < (r + 1) * BR, zsc[...], 0.0)
        acc = jnp.dot(a_ref[...], zfull, preferred_element_type=jnp.float32)

        nrr = pl.num_programs(0)

        @pl.when(r == nrr - 1)
        def _finalize_last():
            p_ref[...] = acc * dinv + dinv * z

        @pl.when(r != nrr - 1)
        def _partial():
            p_ref[...] = acc

    P, Dinv, Z = pl.pallas_call(
        sweep,
        grid=(nr,),
        in_specs=[
            pl.BlockSpec((BR, B * C_IN), lambda r: (r, 0)),
            pl.BlockSpec((C_IN, C_OUT), lambda r: (0, 0)),
            pl.BlockSpec((BR, N), lambda r: (r, 0)),
        ],
        out_specs=[
            pl.BlockSpec((BR, F), lambda r: (r, 0)),
            pl.BlockSpec((BR, 1), lambda r: (r, 0)),
            pl.BlockSpec((BR, F), lambda r: (r, 0)),
        ],
        out_shape=[
            jax.ShapeDtypeStruct((N, F), jnp.float32),
            jax.ShapeDtypeStruct((N, 1), jnp.float32),
            jax.ShapeDtypeStruct((N, F), jnp.float32),
        ],
        scratch_shapes=[pltpu.VMEM((N, F), jnp.float32)],
        compiler_params=pltpu.CompilerParams(
            dimension_semantics=("arbitrary",),
        ),
    )(Xr, W, A)

    def upper(rt, ct, ft, lt, p_ref, dinv_ref, zr_ref, zc_ref, a_ref, o_ref):
        k = pl.program_id(0)
        r = rt[k]
        cb = ct[k]
        zc = zc_ref[...]
        col_ids = jax.lax.broadcasted_iota(jnp.int32, zc.shape, 0) + cb * BC
        zm = jnp.where(col_ids >= (r + 1) * BR, zc, 0.0)
        part = jnp.dot(a_ref[...], zm, preferred_element_type=jnp.float32)

        @pl.when(ft[k] == 1)
        def _first():
            o_ref[...] = p_ref[...] + part

        @pl.when(ft[k] != 1)
        def _acc():
            o_ref[...] = o_ref[...] + part

        @pl.when(lt[k] == 1)
        def _last():
            dinv = dinv_ref[...]
            o_ref[...] = o_ref[...] * dinv + dinv * zr_ref[...]

    Ofull = pl.pallas_call(
        upper,
        grid_spec=pltpu.PrefetchScalarGridSpec(
            num_scalar_prefetch=4,
            grid=(nsteps2,),
            in_specs=[
                pl.BlockSpec((BR, F), lambda k, rt, ct, ft, lt: (rt[k], 0)),
                pl.BlockSpec((BR, 1), lambda k, rt, ct, ft, lt: (rt[k], 0)),
                pl.BlockSpec((BR, F), lambda k, rt, ct, ft, lt: (rt[k], 0)),
                pl.BlockSpec((BC, F), lambda k, rt, ct, ft, lt: (ct[k], 0)),
                pl.BlockSpec((BR, BC), lambda k, rt, ct, ft, lt: (rt[k], ct[k])),
            ],
            out_specs=pl.BlockSpec((BR, F), lambda k, rt, ct, ft, lt: (rt[k], 0)),
        ),
        out_shape=jax.ShapeDtypeStruct((N, F), jnp.float32),
        compiler_params=pltpu.CompilerParams(
            dimension_semantics=("arbitrary",),
        ),
    )(r_tab, cb_tab, first_tab, last_tab, P, Dinv, Z, Z, A)

    nv = nvis * BR
    out2 = jnp.concatenate([Ofull[:nv], P[nv:]], axis=0)
    return out2.reshape(N, B, C_OUT).transpose(1, 0, 2)
